# baseline (speedup 1.0000x reference)
import jax
import jax.numpy as jnp
from jax.experimental import pallas as pl
from jax.experimental.pallas import tpu as pltpu

BM = 1024
BITS = 6
NBUF = 3
NSUB = 4
SUB = BM // NSUB


def _lsh_kernel(x_hbm, w_ref, gates_ref, dec_ref, buf, sems):
    i = pl.program_id(0)
    nsteps = pl.num_programs(0)

    def sub_copy(block, slot, s):
        return pltpu.make_async_copy(
            x_hbm.at[pl.ds(block * BM + s * SUB, SUB), :],
            buf.at[slot, pl.ds(s * SUB, SUB), :],
            sems.at[slot, s],
        )

    def copy_in(block, slot):
        for s in range(NSUB):
            sub_copy(block, slot, s).start()

    @pl.when(i == 0)
    def _():
        for b in range(NBUF):
            copy_in(b, b)

    slot = jax.lax.rem(i, NBUF)
    for s in range(NSUB):
        sub_copy(i, slot, s).wait()

    xb = buf[slot].astype(jnp.bfloat16)
    wb = w_ref[...].astype(jnp.bfloat16)
    h = jax.lax.dot_general(
        xb, wb, (((1,), (0,)), ((), ())),
        preferred_element_type=jnp.float32,
    )
    powers = (1 << jnp.arange(BITS - 1, -1, -1, dtype=jnp.int32)).astype(
        jnp.float32)
    dec = jnp.sum(jnp.where(h > 0, powers[None, :], 0.0), axis=1)
    dec_ref[...] = dec
    gates_ref[...] = jnp.ones_like(dec)

    @pl.when(i + NBUF < nsteps)
    def _():
        copy_in(i + NBUF, slot)


def kernel(x, W):
    n, d = x.shape
    grid = (n // BM,)
    gates, dec = pl.pallas_call(
        _lsh_kernel,
        grid=grid,
        in_specs=[
            pl.BlockSpec(memory_space=pltpu.MemorySpace.HBM),
            pl.BlockSpec((d, BITS), lambda i: (0, 0)),
        ],
        out_specs=[
            pl.BlockSpec((BM,), lambda i: (i,)),
            pl.BlockSpec((BM,), lambda i: (i,)),
        ],
        out_shape=[
            jax.ShapeDtypeStruct((n,), jnp.float32),
            jax.ShapeDtypeStruct((n,), jnp.float32),
        ],
        scratch_shapes=[
            pltpu.VMEM((NBUF, BM, d), jnp.float32),
            pltpu.SemaphoreType.DMA((NBUF, NSUB)),
        ],
    )(x, W)
    return gates, dec


# speedup vs baseline: 1.0562x; 1.0562x over previous
import jax
import jax.numpy as jnp
from jax.experimental import pallas as pl

BM = 1024
BITS = 6


def _lsh_kernel(x_ref, w_ref, gates_ref, dec_ref):
    xb = x_ref[...].astype(jnp.bfloat16)
    wb = w_ref[...].astype(jnp.bfloat16)
    ht = jax.lax.dot_general(
        wb, xb, (((0,), (1,)), ((), ())),
        preferred_element_type=jnp.float32,
    )
    powers = (1 << jnp.arange(BITS - 1, -1, -1, dtype=jnp.int32)).astype(
        jnp.float32)[:, None]
    dec = jnp.sum(jnp.where(ht > 0, powers, 0.0), axis=0)
    dec_ref[...] = dec
    gates_ref[...] = jnp.ones_like(dec)


def kernel(x, W):
    n, d = x.shape
    grid = (n // BM,)
    gates, dec = pl.pallas_call(
        _lsh_kernel,
        grid=grid,
        in_specs=[
            pl.BlockSpec((BM, d), lambda i: (i, 0)),
            pl.BlockSpec((d, BITS), lambda i: (0, 0)),
        ],
        out_specs=[
            pl.BlockSpec((BM,), lambda i: (i,)),
            pl.BlockSpec((BM,), lambda i: (i,)),
        ],
        out_shape=[
            jax.ShapeDtypeStruct((n,), jnp.float32),
            jax.ShapeDtypeStruct((n,), jnp.float32),
        ],
    )(x, W)
    return gates, dec
